# baseline (speedup 1.0000x reference)
import jax
import jax.numpy as jnp
from jax.experimental import pallas as pl
from jax.experimental.pallas import tpu as pltpu

_DIAG_W = 0.707


def _shl_e(t):
    return jnp.concatenate([t[:, :1], t[:, :-1]], axis=1)


def _shr_e(t):
    return jnp.concatenate([t[:, 1:], t[:, -1:]], axis=1)


def _shl_z(t):
    z = jnp.zeros((t.shape[0], 1), t.dtype)
    return jnp.concatenate([z, t[:, :-1]], axis=1)


def _shr_z(t):
    z = jnp.zeros((t.shape[0], 1), t.dtype)
    return jnp.concatenate([t[:, 1:], z], axis=1)


def _nsml_strip(ref, r0, r1, h, w):
    a = max(r0 - 1, 0)
    b = min(r1 + 1, h)
    lo = a - 1
    hi = b + 1
    parts = []
    if lo < 0:
        parts.append(ref[0, 0:1, :])
    parts.append(ref[0, max(lo, 0):min(hi, h), :])
    if hi > h:
        parts.append(ref[0, h - 1:h, :])
    x = jnp.concatenate(parts, axis=0) if len(parts) > 1 else parts[0]

    xsl = _shl_e(x)
    xsr = _shr_e(x)
    dy = jnp.abs(x[1:] - x[:-1])
    dxt = jnp.abs(x - xsl)
    dx = dxt[1:-1]
    d1 = jnp.abs(x[1:] - xsl[:-1])
    d2 = jnp.abs(x[1:] - xsr[:-1])
    t_dr = jnp.concatenate([d1[1:, 1:], dy[1:, -1:]], axis=1)
    t_dl = jnp.concatenate([dy[1:, :1], d2[1:, :-1]], axis=1)
    ortho = (dy[:-1] + dy[1:]) + (dx + _shr_z(dx))
    diag = (d1[:-1] + t_dr) + (d2[:-1] + t_dl)
    s = ortho + _DIAG_W * diag
    sq = s * s

    zrow = jnp.zeros((1, w), jnp.float32)
    ps = []
    if r0 == 0:
        ps.append(zrow)
    ps.append(sq)
    if r1 == h:
        ps.append(zrow)
    sqe = jnp.concatenate(ps, axis=0) if len(ps) > 1 else ps[0]

    m = r1 - r0
    top = sqe[0:m]
    mid = sqe[1:m + 1]
    bot = sqe[2:m + 2]
    v = top + 2.0 * mid + bot
    return _shl_z(v) + 2.0 * v + _shr_z(v)


def _clar_loss(a3, b3, *, interpret=False):
    n, h, w = a3.shape
    strip = h // 4 if h % 4 == 0 else h
    inv_count = 1.0 / (float(n * h * w) * 4096.0)

    def body(a_ref, b_ref, o_ref, acc_ref):
        i = pl.program_id(0)

        @pl.when(i == 0)
        def _():
            acc_ref[...] = jnp.zeros_like(acc_ref)

        part = jnp.zeros((1, w), jnp.float32)
        for r0 in range(0, h, strip):
            na = _nsml_strip(a_ref, r0, r0 + strip, h, w)
            nb = _nsml_strip(b_ref, r0, r0 + strip, h, w)
            d = na - nb
            part = part + jnp.sum(d * d, axis=0, keepdims=True)
        acc_ref[...] += part

        @pl.when(i == n - 1)
        def _():
            total = jnp.sum(acc_ref[...]) * inv_count
            o_ref[...] = jnp.full((1, 1, w), total, jnp.float32)

    out = pl.pallas_call(
        body,
        grid=(n,),
        in_specs=[
            pl.BlockSpec((1, h, w), lambda i: (i, 0, 0)),
            pl.BlockSpec((1, h, w), lambda i: (i, 0, 0)),
        ],
        out_specs=pl.BlockSpec((1, 1, w), lambda i: (0, 0, 0)),
        out_shape=jax.ShapeDtypeStruct((1, 1, w), jnp.float32),
        scratch_shapes=[pltpu.VMEM((1, w), jnp.float32)],
        compiler_params=pltpu.CompilerParams(
            dimension_semantics=("arbitrary",),
            vmem_limit_bytes=48 * 1024 * 1024,
        ),
        name="clar_loss",
        interpret=interpret,
    )(a3, b3)
    return out[0, 0, 0]


def kernel(TensorA, TensorB):
    bsz, c, h, w = TensorA.shape
    a3 = TensorA.reshape(bsz * c, h, w)
    b3 = TensorB.reshape(bsz * c, h, w)
    return _clar_loss(a3, b3)


# speedup vs baseline: 1.2976x; 1.2976x over previous
import jax
import jax.numpy as jnp
from jax.experimental import pallas as pl
from jax.experimental.pallas import tpu as pltpu

_DIAG_W = 0.707
_PAD = 8


def _shl_e(t):
    return jnp.concatenate([t[:, :1], t[:, :-1]], axis=1)


def _shr_e(t):
    return jnp.concatenate([t[:, 1:], t[:, -1:]], axis=1)


def _shl_z(t):
    z = jnp.zeros((t.shape[0], 1), t.dtype)
    return jnp.concatenate([z, t[:, :-1]], axis=1)


def _shr_z(t):
    z = jnp.zeros((t.shape[0], 1), t.dtype)
    return jnp.concatenate([t[:, 1:], z], axis=1)


def _clar_loss(a3, b3, *, interpret=False):
    n, h, w = a3.shape
    strip = h // 4 if h % 4 == 0 else h
    inv_count = 1.0 / (float(n * h * w) * 4096.0)
    p = _PAD

    def body(a_ref, b_ref, o_ref, acc_ref, xp, xsl, xsr, sqd):
        i = pl.program_id(0)

        @pl.when(i == 0)
        def _():
            acc_ref[...] = jnp.zeros_like(acc_ref)

        zrow = jnp.zeros((1, w), jnp.float32)
        sqd[p - 1:p, :] = zrow
        sqd[h + p:h + p + 1, :] = zrow

        for first, ref in ((True, a_ref), (False, b_ref)):
            for r0 in range(0, h, strip):
                v = ref[0, r0:r0 + strip, :]
                xp[p + r0:p + r0 + strip, :] = v
                xsl[p + r0:p + r0 + strip, :] = _shl_e(v)
                xsr[p + r0:p + r0 + strip, :] = _shr_e(v)
            et = ref[0, 0:1, :]
            eb = ref[0, h - 1:h, :]
            xp[p - 1:p, :] = et
            xp[h + p:h + p + 1, :] = eb
            xsl[p - 1:p, :] = _shl_e(et)
            xsl[h + p:h + p + 1, :] = _shl_e(eb)
            xsr[p - 1:p, :] = _shr_e(et)
            xsr[h + p:h + p + 1, :] = _shr_e(eb)

            for r0 in range(0, h, strip):
                b0 = p + r0
                c0 = xp[b0:b0 + strip, :]
                ortho = (jnp.abs(c0 - xp[b0 - 1:b0 - 1 + strip, :])
                         + jnp.abs(c0 - xp[b0 + 1:b0 + 1 + strip, :])
                         + jnp.abs(c0 - xsl[b0:b0 + strip, :])
                         + jnp.abs(c0 - xsr[b0:b0 + strip, :]))
                diag = (jnp.abs(c0 - xsl[b0 - 1:b0 - 1 + strip, :])
                        + jnp.abs(c0 - xsr[b0 - 1:b0 - 1 + strip, :])
                        + jnp.abs(c0 - xsl[b0 + 1:b0 + 1 + strip, :])
                        + jnp.abs(c0 - xsr[b0 + 1:b0 + 1 + strip, :]))
                s = ortho + _DIAG_W * diag
                if first:
                    sqd[b0:b0 + strip, :] = s * s
                else:
                    sqd[b0:b0 + strip, :] = sqd[b0:b0 + strip, :] - s * s

        part = jnp.zeros((1, w), jnp.float32)
        for r0 in range(0, h, strip):
            b0 = p + r0
            v = (sqd[b0 - 1:b0 - 1 + strip, :]
                 + sqd[b0 + 1:b0 + 1 + strip, :]
                 + 2.0 * sqd[b0:b0 + strip, :])
            d = (_shl_z(v) + _shr_z(v)) + 2.0 * v
            part = part + jnp.sum(d * d, axis=0, keepdims=True)
        acc_ref[...] += part

        @pl.when(i == n - 1)
        def _():
            total = jnp.sum(acc_ref[...]) * inv_count
            o_ref[...] = jnp.full((1, 1, w), total, jnp.float32)

    out = pl.pallas_call(
        body,
        grid=(n,),
        in_specs=[
            pl.BlockSpec((1, h, w), lambda i: (i, 0, 0)),
            pl.BlockSpec((1, h, w), lambda i: (i, 0, 0)),
        ],
        out_specs=pl.BlockSpec((1, 1, w), lambda i: (0, 0, 0)),
        out_shape=jax.ShapeDtypeStruct((1, 1, w), jnp.float32),
        scratch_shapes=[
            pltpu.VMEM((1, w), jnp.float32),
            pltpu.VMEM((h + 2 * p, w), jnp.float32),
            pltpu.VMEM((h + 2 * p, w), jnp.float32),
            pltpu.VMEM((h + 2 * p, w), jnp.float32),
            pltpu.VMEM((h + 2 * p, w), jnp.float32),
        ],
        compiler_params=pltpu.CompilerParams(
            dimension_semantics=("arbitrary",),
            vmem_limit_bytes=48 * 1024 * 1024,
        ),
        name="clar_loss",
        interpret=interpret,
    )(a3, b3)
    return out[0, 0, 0]


def kernel(TensorA, TensorB):
    bsz, c, h, w = TensorA.shape
    a3 = TensorA.reshape(bsz * c, h, w)
    b3 = TensorB.reshape(bsz * c, h, w)
    return _clar_loss(a3, b3)


# speedup vs baseline: 1.3231x; 1.0197x over previous
import jax
import jax.numpy as jnp
from jax.experimental import pallas as pl
from jax.experimental.pallas import tpu as pltpu

_DIAG_W = 0.707
_PAD = 8


def _shl_e(t):
    return jnp.concatenate([t[:, :1], t[:, :-1]], axis=1)


def _shr_e(t):
    return jnp.concatenate([t[:, 1:], t[:, -1:]], axis=1)


def _shl_z(t):
    z = jnp.zeros((t.shape[0], 1), t.dtype)
    return jnp.concatenate([z, t[:, :-1]], axis=1)


def _shr_z(t):
    z = jnp.zeros((t.shape[0], 1), t.dtype)
    return jnp.concatenate([t[:, 1:], z], axis=1)


def _clar_loss(a3, b3, *, interpret=False):
    n, h, w = a3.shape
    strip = h // 4 if h % 4 == 0 else h
    inv_count = 1.0 / (float(n * h * w) * 4096.0)
    p = _PAD

    def body(a_ref, b_ref, o_ref, acc_ref, xp, sqd):
        i = pl.program_id(0)

        @pl.when(i == 0)
        def _():
            acc_ref[...] = jnp.zeros_like(acc_ref)

        zrow = jnp.zeros((1, w), jnp.float32)
        sqd[p - 1:p, :] = zrow
        sqd[h + p:h + p + 1, :] = zrow

        for first, ref in ((True, a_ref), (False, b_ref)):
            for r0 in range(0, h, strip):
                xp[p + r0:p + r0 + strip, :] = ref[0, r0:r0 + strip, :]
            xp[p - 1:p, :] = ref[0, 0:1, :]
            xp[h + p:h + p + 1, :] = ref[0, h - 1:h, :]

            for r0 in range(0, h, strip):
                b0 = p + r0
                c0 = xp[b0:b0 + strip, :]
                u0 = xp[b0 - 1:b0 - 1 + strip, :]
                d0 = xp[b0 + 1:b0 + 1 + strip, :]
                ortho = (jnp.abs(c0 - u0) + jnp.abs(c0 - d0)
                         + jnp.abs(c0 - _shl_e(c0))
                         + jnp.abs(c0 - _shr_e(c0)))
                diag = (jnp.abs(c0 - _shl_e(u0)) + jnp.abs(c0 - _shr_e(u0))
                        + jnp.abs(c0 - _shl_e(d0)) + jnp.abs(c0 - _shr_e(d0)))
                s = ortho + _DIAG_W * diag
                if first:
                    sqd[b0:b0 + strip, :] = s * s
                else:
                    sqd[b0:b0 + strip, :] = sqd[b0:b0 + strip, :] - s * s

        part = jnp.zeros((1, w), jnp.float32)
        for r0 in range(0, h, strip):
            b0 = p + r0
            v = (sqd[b0 - 1:b0 - 1 + strip, :]
                 + sqd[b0 + 1:b0 + 1 + strip, :]
                 + 2.0 * sqd[b0:b0 + strip, :])
            d = (_shl_z(v) + _shr_z(v)) + 2.0 * v
            part = part + jnp.sum(d * d, axis=0, keepdims=True)
        acc_ref[...] += part

        @pl.when(i == n - 1)
        def _():
            total = jnp.sum(acc_ref[...]) * inv_count
            o_ref[...] = jnp.full((1, 1, w), total, jnp.float32)

    out = pl.pallas_call(
        body,
        grid=(n,),
        in_specs=[
            pl.BlockSpec((1, h, w), lambda i: (i, 0, 0)),
            pl.BlockSpec((1, h, w), lambda i: (i, 0, 0)),
        ],
        out_specs=pl.BlockSpec((1, 1, w), lambda i: (0, 0, 0)),
        out_shape=jax.ShapeDtypeStruct((1, 1, w), jnp.float32),
        scratch_shapes=[
            pltpu.VMEM((1, w), jnp.float32),
            pltpu.VMEM((h + 2 * p, w), jnp.float32),
            pltpu.VMEM((h + 2 * p, w), jnp.float32),
        ],
        compiler_params=pltpu.CompilerParams(
            dimension_semantics=("arbitrary",),
            vmem_limit_bytes=48 * 1024 * 1024,
        ),
        name="clar_loss",
        interpret=interpret,
    )(a3, b3)
    return out[0, 0, 0]


def kernel(TensorA, TensorB):
    bsz, c, h, w = TensorA.shape
    a3 = TensorA.reshape(bsz * c, h, w)
    b3 = TensorB.reshape(bsz * c, h, w)
    return _clar_loss(a3, b3)


# speedup vs baseline: 1.3567x; 1.0253x over previous
import jax
import jax.numpy as jnp
from jax.experimental import pallas as pl
from jax.experimental.pallas import tpu as pltpu

_DIAG_W = 0.707
_PAD = 8


def _shl_e(t):
    return jnp.concatenate([t[:, :1], t[:, :-1]], axis=1)


def _shr_e(t):
    return jnp.concatenate([t[:, 1:], t[:, -1:]], axis=1)


def _shl_z(t):
    z = jnp.zeros((t.shape[0], 1), t.dtype)
    return jnp.concatenate([z, t[:, :-1]], axis=1)


def _shr_z(t):
    z = jnp.zeros((t.shape[0], 1), t.dtype)
    return jnp.concatenate([t[:, 1:], z], axis=1)


def _clar_loss(a3, b3, *, interpret=False):
    n, h, w = a3.shape
    strip = h // 4 if h % 4 == 0 else h
    inv_count = 1.0 / (float(n * h * w) * 4096.0)
    p = _PAD

    def body(a_ref, b_ref, o_ref, acc_ref, xp, sqd):
        i = pl.program_id(0)

        @pl.when(i == 0)
        def _():
            acc_ref[...] = jnp.zeros_like(acc_ref)

        zrow = jnp.zeros((1, w), jnp.float32)
        sqd[p - 1:p, :] = zrow
        sqd[h + p:h + p + 1, :] = zrow

        for first, ref in ((True, a_ref), (False, b_ref)):
            for r0 in range(0, h, strip):
                xp[p + r0:p + r0 + strip, :] = ref[0, r0:r0 + strip, :]
            xp[p - 1:p, :] = ref[0, 0:1, :]
            xp[h + p:h + p + 1, :] = ref[0, h - 1:h, :]

            for r0 in range(0, h, strip):
                b0 = p + r0
                c0 = xp[b0:b0 + strip, :]
                u0 = xp[b0 - 1:b0 - 1 + strip, :]
                d0 = xp[b0 + 1:b0 + 1 + strip, :]
                cl = _shl_e(c0)
                cr = _shr_e(c0)
                tud = jnp.abs(c0 - u0) + jnp.abs(c0 - d0)
                ortho = tud + (jnp.abs(c0 - cl) + jnp.abs(c0 - cr))
                q = jnp.abs(cr - u0) + jnp.abs(cr - d0)
                r = jnp.abs(cl - u0) + jnp.abs(cl - d0)
                diag = (jnp.concatenate([tud[:, :1], q[:, :-1]], axis=1)
                        + jnp.concatenate([r[:, 1:], tud[:, -1:]], axis=1))
                s = ortho + _DIAG_W * diag
                if first:
                    sqd[b0:b0 + strip, :] = s * s
                else:
                    sqd[b0:b0 + strip, :] = sqd[b0:b0 + strip, :] - s * s

        part = jnp.zeros((1, w), jnp.float32)
        for r0 in range(0, h, strip):
            b0 = p + r0
            v = (sqd[b0 - 1:b0 - 1 + strip, :]
                 + sqd[b0 + 1:b0 + 1 + strip, :]
                 + 2.0 * sqd[b0:b0 + strip, :])
            d = (_shl_z(v) + _shr_z(v)) + 2.0 * v
            part = part + jnp.sum(d * d, axis=0, keepdims=True)
        acc_ref[...] += part

        @pl.when(i == n - 1)
        def _():
            total = jnp.sum(acc_ref[...]) * inv_count
            o_ref[...] = jnp.full((1, 1, w), total, jnp.float32)

    out = pl.pallas_call(
        body,
        grid=(n,),
        in_specs=[
            pl.BlockSpec((1, h, w), lambda i: (i, 0, 0)),
            pl.BlockSpec((1, h, w), lambda i: (i, 0, 0)),
        ],
        out_specs=pl.BlockSpec((1, 1, w), lambda i: (0, 0, 0)),
        out_shape=jax.ShapeDtypeStruct((1, 1, w), jnp.float32),
        scratch_shapes=[
            pltpu.VMEM((1, w), jnp.float32),
            pltpu.VMEM((h + 2 * p, w), jnp.float32),
            pltpu.VMEM((h + 2 * p, w), jnp.float32),
        ],
        compiler_params=pltpu.CompilerParams(
            dimension_semantics=("arbitrary",),
            vmem_limit_bytes=48 * 1024 * 1024,
        ),
        name="clar_loss",
        interpret=interpret,
    )(a3, b3)
    return out[0, 0, 0]


def kernel(TensorA, TensorB):
    bsz, c, h, w = TensorA.shape
    a3 = TensorA.reshape(bsz * c, h, w)
    b3 = TensorB.reshape(bsz * c, h, w)
    return _clar_loss(a3, b3)


# speedup vs baseline: 1.3637x; 1.0052x over previous
import jax
import jax.numpy as jnp
from jax.experimental import pallas as pl
from jax.experimental.pallas import tpu as pltpu

_DIAG_W = 0.707
_PAD = 8


def _shl_e(t):
    return jnp.concatenate([t[:, :1], t[:, :-1]], axis=1)


def _shr_e(t):
    return jnp.concatenate([t[:, 1:], t[:, -1:]], axis=1)


def _shl_z(t):
    z = jnp.zeros((t.shape[0], 1), t.dtype)
    return jnp.concatenate([z, t[:, :-1]], axis=1)


def _shr_z(t):
    z = jnp.zeros((t.shape[0], 1), t.dtype)
    return jnp.concatenate([t[:, 1:], z], axis=1)


def _clar_loss(a3, b3, *, interpret=False):
    n, h, w = a3.shape
    strip = h // 4 if h % 4 == 0 else h
    inv_count = 1.0 / (float(n * h * w) * 4096.0)
    p = _PAD

    def body(a_ref, b_ref, o_ref, acc_ref, xpa, xpb, sqd):
        i = pl.program_id(0)

        @pl.when(i == 0)
        def _():
            acc_ref[...] = jnp.zeros_like(acc_ref)

        zrow = jnp.zeros((1, w), jnp.float32)
        sqd[p - 1:p, :] = zrow
        sqd[h + p:h + p + 1, :] = zrow

        for xp, ref in ((xpa, a_ref), (xpb, b_ref)):
            for r0 in range(0, h, strip):
                xp[p + r0:p + r0 + strip, :] = ref[0, r0:r0 + strip, :]
            xp[p - 1:p, :] = ref[0, 0:1, :]
            xp[h + p:h + p + 1, :] = ref[0, h - 1:h, :]

        def sml2(xp, b0):
            c0 = xp[b0:b0 + strip, :]
            u0 = xp[b0 - 1:b0 - 1 + strip, :]
            d0 = xp[b0 + 1:b0 + 1 + strip, :]
            cl = _shl_e(c0)
            cr = _shr_e(c0)
            tud = jnp.abs(c0 - u0) + jnp.abs(c0 - d0)
            ortho = tud + (jnp.abs(c0 - cl) + jnp.abs(c0 - cr))
            q = jnp.abs(cr - u0) + jnp.abs(cr - d0)
            r = jnp.abs(cl - u0) + jnp.abs(cl - d0)
            diag = (jnp.concatenate([tud[:, :1], q[:, :-1]], axis=1)
                    + jnp.concatenate([r[:, 1:], tud[:, -1:]], axis=1))
            s = ortho + _DIAG_W * diag
            return s * s

        for r0 in range(0, h, strip):
            b0 = p + r0
            sqd[b0:b0 + strip, :] = sml2(xpa, b0) - sml2(xpb, b0)

        part = jnp.zeros((1, w), jnp.float32)
        for r0 in range(0, h, strip):
            b0 = p + r0
            v = (sqd[b0 - 1:b0 - 1 + strip, :]
                 + sqd[b0 + 1:b0 + 1 + strip, :]
                 + 2.0 * sqd[b0:b0 + strip, :])
            d = (_shl_z(v) + _shr_z(v)) + 2.0 * v
            part = part + jnp.sum(d * d, axis=0, keepdims=True)
        acc_ref[...] += part

        @pl.when(i == n - 1)
        def _():
            total = jnp.sum(acc_ref[...]) * inv_count
            o_ref[...] = jnp.full((1, 1, w), total, jnp.float32)

    out = pl.pallas_call(
        body,
        grid=(n,),
        in_specs=[
            pl.BlockSpec((1, h, w), lambda i: (i, 0, 0)),
            pl.BlockSpec((1, h, w), lambda i: (i, 0, 0)),
        ],
        out_specs=pl.BlockSpec((1, 1, w), lambda i: (0, 0, 0)),
        out_shape=jax.ShapeDtypeStruct((1, 1, w), jnp.float32),
        scratch_shapes=[
            pltpu.VMEM((1, w), jnp.float32),
            pltpu.VMEM((h + 2 * p, w), jnp.float32),
            pltpu.VMEM((h + 2 * p, w), jnp.float32),
            pltpu.VMEM((h + 2 * p, w), jnp.float32),
        ],
        compiler_params=pltpu.CompilerParams(
            dimension_semantics=("arbitrary",),
            vmem_limit_bytes=48 * 1024 * 1024,
        ),
        name="clar_loss",
        interpret=interpret,
    )(a3, b3)
    return out[0, 0, 0]


def kernel(TensorA, TensorB):
    bsz, c, h, w = TensorA.shape
    a3 = TensorA.reshape(bsz * c, h, w)
    b3 = TensorB.reshape(bsz * c, h, w)
    return _clar_loss(a3, b3)
